# baseline (speedup 1.0000x reference)
import functools

import jax
import jax.numpy as jnp
from jax import lax
from jax.experimental import pallas as pl
from jax.experimental.pallas import tpu as pltpu
from jax.experimental.pallas import tpu_sc as plsc

MAXLEN = 2048
EMBED_DIM = 128
BATCH = 4

NUM_CORES = 2
NUM_SUBCORES = 16
NUM_WORKERS = NUM_CORES * NUM_SUBCORES
POS_PER_WORKER = MAXLEN // NUM_WORKERS
ROWS_PER_WORKER = BATCH * POS_PER_WORKER
ROW_UNROLL = 1


def _emb_body(x_hbm, table_hbm, pos_hbm, out_hbm, idx_v, rows_v, pos_v,
              sem_i, sem_p, sem_c0, sem_c1, sem_c2, sem_c3, sem_out):
    c = lax.axis_index("c")
    s = lax.axis_index("s")
    w = s * NUM_CORES + c
    t0 = w * POS_PER_WORKER

    cp_pos = pltpu.async_copy(pos_hbm.at[pl.ds(t0, POS_PER_WORKER)],
                              pos_v, sem_p)

    idx_cps = [
        pltpu.async_copy(x_hbm.at[b, pl.ds(t0, POS_PER_WORKER)],
                         idx_v.at[b], sem_i)
        for b in range(BATCH)
    ]
    sems = (sem_c0, sem_c1, sem_c2, sem_c3)
    gathers = []
    for b in range(BATCH):
        idx_cps[b].wait()
        gathers.append(pltpu.async_copy(
            table_hbm.at[idx_v.at[b]],
            rows_v.at[pl.ds(b * POS_PER_WORKER, POS_PER_WORKER)],
            sems[b]))

    cp_pos.wait()
    for cp in gathers:
        cp.wait()

    def add_rows(r, carry):
        p = lax.rem(r, POS_PER_WORKER)
        for k in range(EMBED_DIM // 16):
            ds16 = pl.ds(k * 16, 16)
            plsc.addupdate(rows_v.at[r, ds16], pos_v[p, ds16])
        return carry

    lax.fori_loop(0, ROWS_PER_WORKER, add_rows, 0)

    out_cps = [
        pltpu.async_copy(
            rows_v.at[pl.ds(b * POS_PER_WORKER, POS_PER_WORKER)],
            out_hbm.at[b, pl.ds(t0, POS_PER_WORKER)],
            sem_out)
        for b in range(BATCH)
    ]
    for cp in out_cps:
        cp.wait()


@jax.jit
def _embed(x, token_table, pos_table):
    mesh = plsc.VectorSubcoreMesh(core_axis_name="c", subcore_axis_name="s")
    run = functools.partial(
        pl.kernel,
        mesh=mesh,
        out_type=jax.ShapeDtypeStruct((BATCH, MAXLEN, EMBED_DIM),
                                      jnp.float32),
        scratch_types=[
            pltpu.VMEM((BATCH, POS_PER_WORKER), jnp.int32),
            pltpu.VMEM((ROWS_PER_WORKER, EMBED_DIM), jnp.float32),
            pltpu.VMEM((POS_PER_WORKER, EMBED_DIM), jnp.float32),
            pltpu.SemaphoreType.DMA,
            pltpu.SemaphoreType.DMA,
            pltpu.SemaphoreType.DMA,
            pltpu.SemaphoreType.DMA,
            pltpu.SemaphoreType.DMA,
            pltpu.SemaphoreType.DMA,
            pltpu.SemaphoreType.DMA,
        ],
    )(_emb_body)
    return run(x, token_table, pos_table)


def kernel(x, token_table, pos_table):
    return _embed(x.astype(jnp.int32), token_table, pos_table)


# speedup vs baseline: 1.2181x; 1.2181x over previous
import functools

import jax
import jax.numpy as jnp
from jax import lax
from jax.experimental import pallas as pl
from jax.experimental.pallas import tpu as pltpu
from jax.experimental.pallas import tpu_sc as plsc

MAXLEN = 2048
EMBED_DIM = 128
BATCH = 4

NUM_CORES = 2
NUM_SUBCORES = 16
NUM_WORKERS = NUM_CORES * NUM_SUBCORES
POS_PER_WORKER = MAXLEN // NUM_WORKERS
ROWS_PER_WORKER = BATCH * POS_PER_WORKER
ROW_UNROLL = 1


def _emb_body(x_hbm, table_hbm, pos_hbm, out_hbm, idx_v, rows_v, pos_v,
              sem_i, sem_p, sem_c0, sem_c1, sem_c2, sem_c3, sem_out):
    c = lax.axis_index("c")
    s = lax.axis_index("s")
    w = s * NUM_CORES + c
    t0 = w * POS_PER_WORKER

    cp_pos = pltpu.async_copy(pos_hbm.at[pl.ds(t0, POS_PER_WORKER)],
                              pos_v, sem_p)

    idx_cps = [
        pltpu.async_copy(x_hbm.at[b, pl.ds(t0, POS_PER_WORKER)],
                         idx_v.at[b], sem_i)
        for b in range(BATCH)
    ]
    sems = (sem_c0, sem_c1, sem_c2, sem_c3)
    gathers = []
    for b in range(BATCH):
        idx_cps[b].wait()
        gathers.append(pltpu.async_copy(
            table_hbm.at[idx_v.at[b]],
            rows_v.at[pl.ds(b * POS_PER_WORKER, POS_PER_WORKER)],
            sems[b]))

    cp_pos.wait()
    out_cps = []
    for b in range(BATCH):
        gathers[b].wait()
        base = b * POS_PER_WORKER

        def add_rows(i, carry, base=base):
            for u in range(ROW_UNROLL):
                r = i * ROW_UNROLL + u
                for k in range(EMBED_DIM // 16):
                    ds16 = pl.ds(k * 16, 16)
                    plsc.addupdate(rows_v.at[base + r, ds16],
                                   pos_v[r, ds16])
            return carry

        lax.fori_loop(0, POS_PER_WORKER // ROW_UNROLL, add_rows, 0)
        out_cps.append(pltpu.async_copy(
            rows_v.at[pl.ds(base, POS_PER_WORKER)],
            out_hbm.at[b, pl.ds(t0, POS_PER_WORKER)],
            sem_out))

    for cp in out_cps:
        cp.wait()


@jax.jit
def _embed(x, token_table, pos_table):
    mesh = plsc.VectorSubcoreMesh(core_axis_name="c", subcore_axis_name="s")
    run = functools.partial(
        pl.kernel,
        mesh=mesh,
        out_type=jax.ShapeDtypeStruct((BATCH, MAXLEN, EMBED_DIM),
                                      jnp.float32),
        scratch_types=[
            pltpu.VMEM((BATCH, POS_PER_WORKER), jnp.int32),
            pltpu.VMEM((ROWS_PER_WORKER, EMBED_DIM), jnp.float32),
            pltpu.VMEM((POS_PER_WORKER, EMBED_DIM), jnp.float32),
            pltpu.SemaphoreType.DMA,
            pltpu.SemaphoreType.DMA,
            pltpu.SemaphoreType.DMA,
            pltpu.SemaphoreType.DMA,
            pltpu.SemaphoreType.DMA,
            pltpu.SemaphoreType.DMA,
            pltpu.SemaphoreType.DMA,
        ],
    )(_emb_body)
    return run(x, token_table, pos_table)


def kernel(x, token_table, pos_table):
    return _embed(x.astype(jnp.int32), token_table, pos_table)


# speedup vs baseline: 1.2481x; 1.0246x over previous
import functools

import jax
import jax.numpy as jnp
from jax import lax
from jax.experimental import pallas as pl
from jax.experimental.pallas import tpu as pltpu
from jax.experimental.pallas import tpu_sc as plsc

MAXLEN = 2048
EMBED_DIM = 128
BATCH = 4

NUM_CORES = 2
NUM_SUBCORES = 16
NUM_WORKERS = NUM_CORES * NUM_SUBCORES
POS_PER_WORKER = MAXLEN // NUM_WORKERS
ROWS_PER_WORKER = BATCH * POS_PER_WORKER
NPAIR = BATCH // 2


def _emb_body(x_hbm, table_hbm, pos_hbm, out_hbm, idx_v, rows_v, pos_v,
              sem_i, sem_p, sem_c0, sem_c1, sem_out):
    c = lax.axis_index("c")
    s = lax.axis_index("s")
    w = s * NUM_CORES + c
    t0 = w * POS_PER_WORKER

    cp_pos = pltpu.async_copy(pos_hbm.at[pl.ds(t0, POS_PER_WORKER)],
                              pos_v, sem_p)

    idx_cps = [
        pltpu.async_copy(x_hbm.at[b, pl.ds(t0, POS_PER_WORKER)],
                         idx_v.at[b // 2,
                                  pl.ds((b % 2) * POS_PER_WORKER,
                                        POS_PER_WORKER)],
                         sem_i)
        for b in range(BATCH)
    ]
    sems = (sem_c0, sem_c1)
    gathers = []
    for j in range(NPAIR):
        idx_cps[2 * j].wait()
        idx_cps[2 * j + 1].wait()
        gathers.append(pltpu.async_copy(
            table_hbm.at[idx_v.at[j]],
            rows_v.at[pl.ds(j * 2 * POS_PER_WORKER, 2 * POS_PER_WORKER)],
            sems[j]))

    cp_pos.wait()
    out_cps = []
    for j in range(NPAIR):
        gathers[j].wait()
        base = j * 2 * POS_PER_WORKER

        def add_rows(r, carry, base=base):
            for k in range(EMBED_DIM // 16):
                ds16 = pl.ds(k * 16, 16)
                p = pos_v[r, ds16]
                plsc.addupdate(rows_v.at[base + r, ds16], p)
                plsc.addupdate(rows_v.at[base + POS_PER_WORKER + r, ds16],
                               p)
            return carry

        lax.fori_loop(0, POS_PER_WORKER, add_rows, 0)
        for h in range(2):
            out_cps.append(pltpu.async_copy(
                rows_v.at[pl.ds(base + h * POS_PER_WORKER,
                                POS_PER_WORKER)],
                out_hbm.at[2 * j + h, pl.ds(t0, POS_PER_WORKER)],
                sem_out))

    for cp in out_cps:
        cp.wait()


@jax.jit
def _embed(x, token_table, pos_table):
    mesh = plsc.VectorSubcoreMesh(core_axis_name="c", subcore_axis_name="s")
    run = functools.partial(
        pl.kernel,
        mesh=mesh,
        out_type=jax.ShapeDtypeStruct((BATCH, MAXLEN, EMBED_DIM),
                                      jnp.float32),
        scratch_types=[
            pltpu.VMEM((NPAIR, 2 * POS_PER_WORKER), jnp.int32),
            pltpu.VMEM((ROWS_PER_WORKER, EMBED_DIM), jnp.float32),
            pltpu.VMEM((POS_PER_WORKER, EMBED_DIM), jnp.float32),
            pltpu.SemaphoreType.DMA,
            pltpu.SemaphoreType.DMA,
            pltpu.SemaphoreType.DMA,
            pltpu.SemaphoreType.DMA,
            pltpu.SemaphoreType.DMA,
        ],
    )(_emb_body)
    return run(x, token_table, pos_table)


def kernel(x, token_table, pos_table):
    return _embed(x.astype(jnp.int32), token_table, pos_table)
